# baseline (speedup 1.0000x reference)
import functools

import jax
import jax.numpy as jnp
from jax import lax
from jax.experimental import pallas as pl
from jax.experimental.pallas import tpu as pltpu
from jax.experimental.pallas import tpu_sc as plsc

D_MODEL = 64
RANK = 32
SCALING = 16.0 / 32.0

try:
    _info = plsc.get_sparse_core_info()
    _NC, _NS = _info.num_cores, _info.num_subcores
except Exception:
    _NC, _NS = 2, 16
_NW = _NC * _NS


def _fuse_body(blk, base_t_ref, a_t_ref, b_t_ref, out_ref):
    f_t = base_t_ref[...] + SCALING * jnp.dot(
        b_t_ref[...], a_t_ref[...], preferred_element_type=jnp.float32
    )
    out_ref[:, 0:64] = jnp.transpose(f_t)


@functools.lru_cache(maxsize=None)
def _make_fuse(vocab, rank, d_model, blk):
    grid = (vocab + blk - 1) // blk
    return pl.pallas_call(
        functools.partial(_fuse_body, blk),
        grid=(grid,),
        in_specs=[
            pl.BlockSpec((d_model, blk), lambda i: (0, i)),
            pl.BlockSpec((rank, blk), lambda i: (0, i)),
            pl.BlockSpec((d_model, rank), lambda i: (0, 0)),
        ],
        out_specs=pl.BlockSpec((blk, 2 * d_model), lambda i: (i, 0)),
        out_shape=jax.ShapeDtypeStruct((vocab, 2 * d_model), jnp.float32),
    )


@functools.lru_cache(maxsize=None)
def _make_gather(vocab2, d_model, seq, bsz, chunk=128):
    n_tokens = seq * bsz
    n_per_w = n_tokens // _NW
    n_chunks = n_per_w // chunk
    mesh = plsc.VectorSubcoreMesh(core_axis_name="c", subcore_axis_name="s")

    @functools.partial(
        pl.kernel,
        mesh=mesh,
        compiler_params=pltpu.CompilerParams(
            use_tc_tiling_on_sc=False, needs_layout_passes=False
        ),
        out_type=jax.ShapeDtypeStruct(
            (seq, d_model // 8, bsz // chunk, 8, chunk), jnp.float32
        ),
        scratch_types=[
            pltpu.VMEM((n_chunks, chunk), jnp.int32),
            pltpu.VMEM((2, chunk, d_model), jnp.float32),
            pltpu.VMEM((d_model // 8, 8, chunk), jnp.float32),
            pltpu.SemaphoreType.DMA((2,)),
        ],
    )
    def gather(table_hbm, idx_hbm, out_hbm, idx_v, rows_v, rows_t, sem):
        wid = lax.axis_index("s") * _NC + lax.axis_index("c")
        pltpu.sync_copy(idx_hbm.at[wid], idx_v)
        lane = lax.iota(jnp.int32, 16)

        pltpu.async_copy(table_hbm.at[idx_v.at[0]], rows_v.at[0], sem.at[0])

        def body(j, carry):
            buf = lax.rem(j, 2)

            @pl.when(j + 1 < n_chunks)
            def _():
                pltpu.async_copy(
                    table_hbm.at[idx_v.at[j + 1]],
                    rows_v.at[1 - buf],
                    sem.at[1 - buf],
                )

            pltpu.make_async_copy(
                table_hbm.at[idx_v.at[j]], rows_v.at[buf], sem.at[buf]
            ).wait()

            def tr(d, c):
                dt = lax.shift_right_logical(d, 3)
                dr = lax.rem(d, 8)
                cidx = jnp.full((16,), 0, jnp.int32) + d
                for b16 in range(chunk // 16):
                    v = plsc.load_gather(
                        rows_v.at[buf], [b16 * 16 + lane, cidx]
                    )
                    rows_t[dt, dr, pl.ds(b16 * 16, 16)] = v
                return c

            lax.fori_loop(0, d_model, tr, 0, unroll=4)

            t0 = (wid * n_chunks + j) * chunk
            l = lax.shift_right_logical(t0, 12)
            bt = lax.shift_right_logical(lax.rem(t0, bsz), 7)
            pltpu.sync_copy(rows_t, out_hbm.at[l, :, bt, :, :])
            return carry

        lax.fori_loop(0, n_chunks, body, 0)

    return gather


def kernel(input_ids, base_table, adapter_A, adapter_B):
    bsz, seq = input_ids.shape
    vocab = base_table.shape[0]
    n_tokens = bsz * seq
    chunk = 128

    fused128 = _make_fuse(vocab, RANK, D_MODEL, 6400)(
        base_table.T, adapter_A.T, adapter_B.T
    )
    fused = fused128.reshape(2 * vocab, D_MODEL)

    n_per_w = n_tokens // _NW
    idx = (input_ids.astype(jnp.int32) * 2).T.reshape(
        _NW, n_per_w // chunk, chunk
    )
    out5 = _make_gather(2 * vocab, D_MODEL, seq, bsz, chunk)(fused, idx)
    return jnp.transpose(out5, (2, 4, 0, 1, 3)).reshape(bsz, seq, D_MODEL)


# speedup vs baseline: 1.6969x; 1.6969x over previous
import functools

import jax
import jax.numpy as jnp
from jax import lax
from jax.experimental import pallas as pl
from jax.experimental.pallas import tpu as pltpu
from jax.experimental.pallas import tpu_sc as plsc

D_MODEL = 64
RANK = 32
SCALING = 16.0 / 32.0

try:
    _info = plsc.get_sparse_core_info()
    _NC, _NS = _info.num_cores, _info.num_subcores
except Exception:
    _NC, _NS = 2, 16
_NW = _NC * _NS


def _fuse_body(blk, base_t_ref, a_t_ref, b_t_ref, out_ref):
    f_t = base_t_ref[...] + SCALING * jnp.dot(
        b_t_ref[...], a_t_ref[...], preferred_element_type=jnp.float32
    )
    out_ref[:, 0:64] = jnp.transpose(f_t)


@functools.lru_cache(maxsize=None)
def _make_fuse(vocab, rank, d_model, blk):
    grid = (vocab + blk - 1) // blk
    return pl.pallas_call(
        functools.partial(_fuse_body, blk),
        grid=(grid,),
        in_specs=[
            pl.BlockSpec((d_model, blk), lambda i: (0, i)),
            pl.BlockSpec((rank, blk), lambda i: (0, i)),
            pl.BlockSpec((d_model, rank), lambda i: (0, 0)),
        ],
        out_specs=pl.BlockSpec((blk, 2 * d_model), lambda i: (i, 0)),
        out_shape=jax.ShapeDtypeStruct((vocab, 2 * d_model), jnp.float32),
    )


@functools.lru_cache(maxsize=None)
def _make_gather(vocab2, d_model, seq, bsz, chunk=128):
    n_tokens = seq * bsz
    n_per_w = n_tokens // _NW
    n_chunks = n_per_w // chunk
    mesh = plsc.VectorSubcoreMesh(core_axis_name="c", subcore_axis_name="s")

    @functools.partial(
        pl.kernel,
        mesh=mesh,
        compiler_params=pltpu.CompilerParams(use_tc_tiling_on_sc=False),
        out_type=jax.ShapeDtypeStruct((n_tokens, d_model), jnp.float32),
        scratch_types=[
            pltpu.VMEM((n_chunks, chunk), jnp.int32),
            pltpu.VMEM((2, chunk, d_model), jnp.float32),
            pltpu.SemaphoreType.DMA((2,)),
        ],
    )
    def gather(table_hbm, idx_hbm, out_hbm, idx_v, rows_v, sem):
        wid = lax.axis_index("s") * _NC + lax.axis_index("c")
        pltpu.sync_copy(idx_hbm.at[wid], idx_v)

        pltpu.async_copy(table_hbm.at[idx_v.at[0]], rows_v.at[0], sem.at[0])

        def body(j, carry):
            buf = lax.rem(j, 2)

            @pl.when(j + 1 < n_chunks)
            def _():
                pltpu.async_copy(
                    table_hbm.at[idx_v.at[j + 1]],
                    rows_v.at[1 - buf],
                    sem.at[1 - buf],
                )

            pltpu.make_async_copy(
                table_hbm.at[idx_v.at[j]], rows_v.at[buf], sem.at[buf]
            ).wait()

            row0 = (wid * n_chunks + j) * chunk
            pltpu.sync_copy(rows_v.at[buf], out_hbm.at[pl.ds(row0, chunk)])
            return carry

        lax.fori_loop(0, n_chunks, body, 0)

    return gather


def kernel(input_ids, base_table, adapter_A, adapter_B):
    bsz, seq = input_ids.shape
    vocab = base_table.shape[0]
    n_tokens = bsz * seq
    chunk = 128

    fused128 = _make_fuse(vocab, RANK, D_MODEL, 6400)(
        base_table.T, adapter_A.T, adapter_B.T
    )
    fused = fused128.reshape(2 * vocab, D_MODEL)

    n_per_w = n_tokens // _NW
    idx = (input_ids.astype(jnp.int32) * 2).T.reshape(
        _NW, n_per_w // chunk, chunk
    )
    out = _make_gather(2 * vocab, D_MODEL, seq, bsz, chunk)(fused, idx)
    return jnp.transpose(out.reshape(seq, bsz, D_MODEL), (1, 0, 2))


# speedup vs baseline: 1.8172x; 1.0709x over previous
import functools

import jax
import jax.numpy as jnp
from jax import lax
from jax.experimental import pallas as pl
from jax.experimental.pallas import tpu as pltpu
from jax.experimental.pallas import tpu_sc as plsc

D_MODEL = 64
RANK = 32
SCALING = 16.0 / 32.0

try:
    _info = plsc.get_sparse_core_info()
    _NC, _NS = _info.num_cores, _info.num_subcores
except Exception:
    _NC, _NS = 2, 16
_NW = _NC * _NS


def _fuse_body(blk, base_t_ref, a_t_ref, b_t_ref, out_ref):
    f_t = base_t_ref[...] + SCALING * jnp.dot(
        b_t_ref[...], a_t_ref[...], preferred_element_type=jnp.float32
    )
    out_ref[:, 0:64] = jnp.transpose(f_t)


@functools.lru_cache(maxsize=None)
def _make_fuse(vocab, rank, d_model, blk):
    grid = (vocab + blk - 1) // blk
    return pl.pallas_call(
        functools.partial(_fuse_body, blk),
        grid=(grid,),
        in_specs=[
            pl.BlockSpec((d_model, blk), lambda i: (0, i)),
            pl.BlockSpec((rank, blk), lambda i: (0, i)),
            pl.BlockSpec((d_model, rank), lambda i: (0, 0)),
        ],
        out_specs=pl.BlockSpec((blk, 2 * d_model), lambda i: (i, 0)),
        out_shape=jax.ShapeDtypeStruct((vocab, 2 * d_model), jnp.float32),
    )


@functools.lru_cache(maxsize=None)
def _make_gather(vocab2, d_model, seq, bsz, chunk=128):
    n_tokens = seq * bsz
    n_per_w = n_tokens // _NW
    n_chunks = n_per_w // chunk
    mesh = plsc.VectorSubcoreMesh(core_axis_name="c", subcore_axis_name="s")

    @functools.partial(
        pl.kernel,
        mesh=mesh,
        compiler_params=pltpu.CompilerParams(
            use_tc_tiling_on_sc=False, needs_layout_passes=False
        ),
        out_type=jax.ShapeDtypeStruct(
            (seq, d_model // 8, bsz // chunk, 8, chunk), jnp.float32
        ),
        scratch_types=[
            pltpu.VMEM((n_chunks, chunk), jnp.int32),
            pltpu.VMEM((2, chunk, d_model), jnp.float32),
            pltpu.VMEM((2, d_model // 8, 8, chunk), jnp.float32),
            pltpu.SemaphoreType.DMA((2,)),
            pltpu.SemaphoreType.DMA((2,)),
        ],
    )
    def gather(table_hbm, idx_hbm, out_hbm, idx_v, rows_v, rows_t, sem, wsem):
        wid = lax.axis_index("s") * _NC + lax.axis_index("c")
        pltpu.sync_copy(idx_hbm.at[wid], idx_v)
        lane = lax.iota(jnp.int32, 16)

        pltpu.async_copy(table_hbm.at[idx_v.at[0]], rows_v.at[0], sem.at[0])

        def out_slice(j):
            t0 = (wid * n_chunks + j) * chunk
            l = lax.shift_right_logical(t0, 12)
            bt = lax.shift_right_logical(lax.rem(t0, bsz), 7)
            return out_hbm.at[l, :, bt, :, :]

        def body(j, carry):
            buf = lax.rem(j, 2)

            @pl.when(j + 1 < n_chunks)
            def _():
                pltpu.async_copy(
                    table_hbm.at[idx_v.at[j + 1]],
                    rows_v.at[1 - buf],
                    sem.at[1 - buf],
                )

            pltpu.make_async_copy(
                table_hbm.at[idx_v.at[j]], rows_v.at[buf], sem.at[buf]
            ).wait()

            @pl.when(j >= 2)
            def _():
                pltpu.make_async_copy(
                    rows_t.at[buf], out_slice(j - 2), wsem.at[buf]
                ).wait()

            for g in range(chunk // 16):
                ridx = g * 16 + lane

                def tr(s, cidx, ridx=ridx):
                    v = plsc.load_gather(rows_v.at[buf], [ridx, cidx])
                    plsc.store_scatter(
                        rows_t.at[buf],
                        [
                            lax.shift_right_logical(cidx, 3),
                            lax.bitwise_and(cidx, 7),
                            ridx,
                        ],
                        v,
                    )
                    return lax.bitwise_and(cidx + 1, d_model - 1)

                lax.fori_loop(0, d_model, tr, lane, unroll=8)

            pltpu.async_copy(rows_t.at[buf], out_slice(j), wsem.at[buf])
            return carry

        lax.fori_loop(0, n_chunks, body, 0)

        pltpu.make_async_copy(
            rows_t.at[lax.rem(n_chunks, 2)],
            out_slice(n_chunks - 2),
            wsem.at[lax.rem(n_chunks, 2)],
        ).wait()
        pltpu.make_async_copy(
            rows_t.at[lax.rem(n_chunks + 1, 2)],
            out_slice(n_chunks - 1),
            wsem.at[lax.rem(n_chunks + 1, 2)],
        ).wait()

    return gather


def kernel(input_ids, base_table, adapter_A, adapter_B):
    bsz, seq = input_ids.shape
    vocab = base_table.shape[0]
    n_tokens = bsz * seq
    chunk = 128

    fused128 = _make_fuse(vocab, RANK, D_MODEL, 6400)(
        base_table.T, adapter_A.T, adapter_B.T
    )
    fused = fused128.reshape(2 * vocab, D_MODEL)

    n_per_w = n_tokens // _NW
    idx = (input_ids.astype(jnp.int32) * 2).T.reshape(
        _NW, n_per_w // chunk, chunk
    )
    out5 = _make_gather(2 * vocab, D_MODEL, seq, bsz, chunk)(fused, idx)
    return jnp.transpose(out5, (2, 4, 0, 1, 3)).reshape(bsz, seq, D_MODEL)
